# Initial kernel scaffold; baseline (speedup 1.0000x reference)
import jax
import jax.numpy as jnp
from jax.experimental import pallas as pl


def kernel(pc, feature, sa1_w0, sa1_g0, sa1_b0, sa1_w1, sa1_g1, sa1_b1, sa1_w2, sa1_g2, sa1_b2, sa2_w0, sa2_g0, sa2_b0, sa2_w1, sa2_g1, sa2_b1, sa2_w2, sa2_g2, sa2_b2, conv1_w, bn1_g, bn1_b, conv2_w, conv2_b):
    raise NotImplementedError("write your pallas kernel here")


# speedup vs baseline: 5.8176x; 5.8176x over previous
import functools

import jax
import jax.numpy as jnp
from jax import lax
from jax.experimental import pallas as pl
from jax.experimental.pallas import tpu as pltpu
from jax.experimental.pallas import tpu_sc as plsc

KNN = 16
EPS = 1e-5
NPTS = 8192
NBATCH = 2
TQ = 8
RROWS = 2048


def _knn_body(q_ref, kall_ref, f_ref, wx_ref, wf_ref,
              idx_ref, p_ref, g_ref, keys_ref):
    b = pl.program_id(0)
    q = q_ref[0]
    kk = kall_ref[0]
    sqk = jnp.sum(kk * kk, axis=0).reshape(1, NPTS)
    sqq = jnp.sum(q * q, axis=1).reshape(TQ, 1)
    dot = lax.dot_general(q, kk, (((1,), (0,)), ((), ())),
                          preferred_element_type=jnp.float32)
    d = jnp.maximum(sqk + sqq - 2.0 * dot, 0.0)
    keys = (lax.bitcast_convert_type(d, jnp.int32) & jnp.int32(-8192)) | \
        lax.broadcasted_iota(jnp.int32, (TQ, NPTS), 1)
    keys_ref[...] = keys
    cols = []
    bound = jnp.full((TQ, 1), -1, jnp.int32)
    for _ in range(KNN):
        masked = jnp.where(keys_ref[...] > bound, keys_ref[...],
                           jnp.int32(2**31 - 1))
        bound = jnp.min(masked, axis=1, keepdims=True)
        cols.append(bound)
    packed = jnp.concatenate(cols, axis=1)
    idx_ref[0] = (packed & jnp.int32(8191)) + b * NPTS
    p1 = lax.dot_general(q, wx_ref[...], (((1,), (0,)), ((), ())),
                         preferred_element_type=jnp.float32)
    g1 = p1 + lax.dot_general(f_ref[0], wf_ref[...], (((1,), (0,)), ((), ())),
                              preferred_element_type=jnp.float32)
    p_ref[0] = p1
    g_ref[0] = g1


def _knn_call(xyzt, pc, featt, wxt, wft, cout):
    nt = NPTS // TQ
    return pl.pallas_call(
        _knn_body,
        grid=(NBATCH, nt),
        in_specs=[
            pl.BlockSpec((1, TQ, 3), lambda b, t: (b, t, 0)),
            pl.BlockSpec((1, 3, NPTS), lambda b, t: (b, 0, 0)),
            pl.BlockSpec((1, TQ, 3), lambda b, t: (b, t, 0)),
            pl.BlockSpec((3, cout), lambda b, t: (0, 0)),
            pl.BlockSpec((3, cout), lambda b, t: (0, 0)),
        ],
        out_specs=[
            pl.BlockSpec((1, TQ, KNN), lambda b, t: (b, t, 0)),
            pl.BlockSpec((1, TQ, cout), lambda b, t: (b, t, 0)),
            pl.BlockSpec((1, TQ, cout), lambda b, t: (b, t, 0)),
        ],
        out_shape=[
            jax.ShapeDtypeStruct((NBATCH, NPTS, KNN), jnp.int32),
            jax.ShapeDtypeStruct((NBATCH, NPTS, cout), jnp.float32),
            jax.ShapeDtypeStruct((NBATCH, NPTS, cout), jnp.float32),
        ],
        scratch_shapes=[pltpu.VMEM((TQ, NPTS), jnp.int32)],
    )(xyzt, pc, featt, wxt, wft)


def _sc_gather(table, idx2d, cwidth):
    rows = idx2d.shape[0] * 128
    nworkers = 32
    per_w = rows // nworkers
    nch = per_w // 128
    mesh = plsc.VectorSubcoreMesh(core_axis_name="c", subcore_axis_name="s")

    @functools.partial(
        pl.kernel,
        mesh=mesh,
        out_type=jax.ShapeDtypeStruct((rows, cwidth), jnp.float32),
        compiler_params=pltpu.CompilerParams(use_tc_tiling_on_sc=False),
        scratch_types=[
            pltpu.VMEM((nch, 128), jnp.int32),
            pltpu.VMEM((128, cwidth), jnp.float32),
            pltpu.SemaphoreType.DMA,
        ],
    )
    def gather_k(table_hbm, idx_hbm, out_hbm, idx_v, rows_v, sem):
        wid = lax.axis_index("s") * 2 + lax.axis_index("c")
        pltpu.sync_copy(idx_hbm.at[pl.ds(wid * nch, nch)], idx_v)

        def step(j, carry):
            pltpu.async_copy(table_hbm.at[idx_v.at[j]], rows_v, sem).wait()
            pltpu.sync_copy(rows_v,
                            out_hbm.at[pl.ds((wid * nch + j) * 128, 128)])
            return carry

        lax.fori_loop(0, nch, step, 0)

    return gather_k(table, idx2d)


def _mlpa_body(g_ref, p_ref, y_ref, t1_ref, t2_ref):
    t = pl.program_id(1)
    g = g_ref[0]
    p = p_ref[0]
    c = g.shape[1]
    y = (g.reshape(RROWS // KNN, KNN, c) - p[:, None, :]).reshape(RROWS, c)
    y_ref[0] = y

    @pl.when(t == 0)
    def _():
        t1_ref[...] = jnp.zeros_like(t1_ref)
        t2_ref[...] = jnp.zeros_like(t2_ref)

    t1_ref[...] += jnp.sum(y, axis=0).reshape(1, 1, -1)
    t2_ref[...] += jnp.sum(y * y, axis=0).reshape(1, 1, -1)


def _mlpa_call(gathered, pmat, c):
    m = NPTS * KNN
    nt = m // RROWS
    return pl.pallas_call(
        _mlpa_body,
        grid=(NBATCH, nt),
        in_specs=[
            pl.BlockSpec((1, RROWS, c), lambda b, t: (b, t, 0)),
            pl.BlockSpec((1, RROWS // KNN, c), lambda b, t: (b, t, 0)),
        ],
        out_specs=[
            pl.BlockSpec((1, RROWS, c), lambda b, t: (b, t, 0)),
            pl.BlockSpec((1, 1, c), lambda b, t: (b, 0, 0)),
            pl.BlockSpec((1, 1, c), lambda b, t: (b, 0, 0)),
        ],
        out_shape=[
            jax.ShapeDtypeStruct((NBATCH, m, c), jnp.float32),
            jax.ShapeDtypeStruct((NBATCH, 1, c), jnp.float32),
            jax.ShapeDtypeStruct((NBATCH, 1, c), jnp.float32),
        ],
    )(gathered, pmat)


def _layer_body(y_ref, s1_ref, s2_ref, gam_ref, bet_ref, w_ref,
                o_ref, t1_ref, t2_ref, *, mcount):
    t = pl.program_id(1)
    mean = s1_ref[0] / mcount
    var = s2_ref[0] / mcount - mean * mean
    scale = gam_ref[...] * lax.rsqrt(var + EPS)
    shift = bet_ref[...] - mean * scale
    h = jnp.maximum(y_ref[0] * scale + shift, 0.0)
    ynew = lax.dot_general(h, w_ref[...], (((1,), (1,)), ((), ())),
                           preferred_element_type=jnp.float32)
    o_ref[0] = ynew

    @pl.when(t == 0)
    def _():
        t1_ref[...] = jnp.zeros_like(t1_ref)
        t2_ref[...] = jnp.zeros_like(t2_ref)

    t1_ref[...] += jnp.sum(ynew, axis=0).reshape(1, 1, -1)
    t2_ref[...] += jnp.sum(ynew * ynew, axis=0).reshape(1, 1, -1)


def _layer_call(y, s1, s2, gam, bet, w):
    m = NPTS * KNN
    nt = m // RROWS
    cin = y.shape[2]
    cout = w.shape[0]
    body = functools.partial(_layer_body, mcount=float(m))
    return pl.pallas_call(
        body,
        grid=(NBATCH, nt),
        in_specs=[
            pl.BlockSpec((1, RROWS, cin), lambda b, t: (b, t, 0)),
            pl.BlockSpec((1, 1, cin), lambda b, t: (b, 0, 0)),
            pl.BlockSpec((1, 1, cin), lambda b, t: (b, 0, 0)),
            pl.BlockSpec((1, cin), lambda b, t: (0, 0)),
            pl.BlockSpec((1, cin), lambda b, t: (0, 0)),
            pl.BlockSpec((cout, cin), lambda b, t: (0, 0)),
        ],
        out_specs=[
            pl.BlockSpec((1, RROWS, cout), lambda b, t: (b, t, 0)),
            pl.BlockSpec((1, 1, cout), lambda b, t: (b, 0, 0)),
            pl.BlockSpec((1, 1, cout), lambda b, t: (b, 0, 0)),
        ],
        out_shape=[
            jax.ShapeDtypeStruct((NBATCH, m, cout), jnp.float32),
            jax.ShapeDtypeStruct((NBATCH, 1, cout), jnp.float32),
            jax.ShapeDtypeStruct((NBATCH, 1, cout), jnp.float32),
        ],
    )(y, s1, s2, gam, bet, w)


def _pool1_body(y_ref, s1_ref, s2_ref, gam_ref, bet_ref, xyz_ref,
                wx_ref, wf_ref, p2_ref, g2_ref, *, mcount):
    mean = s1_ref[0] / mcount
    var = s2_ref[0] / mcount - mean * mean
    scale = gam_ref[...] * lax.rsqrt(var + EPS)
    shift = bet_ref[...] - mean * scale
    h = jnp.maximum(y_ref[0] * scale + shift, 0.0)
    c = h.shape[1]
    f = jnp.max(h.reshape(RROWS // KNN, KNN, c), axis=1)
    p2 = lax.dot_general(xyz_ref[0], wx_ref[...], (((1,), (0,)), ((), ())),
                         preferred_element_type=jnp.float32)
    g2 = p2 + lax.dot_general(f, wf_ref[...], (((1,), (1,)), ((), ())),
                              preferred_element_type=jnp.float32)
    p2_ref[0] = p2
    g2_ref[0] = g2


def _pool1_call(y, s1, s2, gam, bet, xyzt, wxt, wf):
    m = NPTS * KNN
    nt = m // RROWS
    tn = RROWS // KNN
    cin = y.shape[2]
    cout = wf.shape[0]
    body = functools.partial(_pool1_body, mcount=float(m))
    return pl.pallas_call(
        body,
        grid=(NBATCH, nt),
        in_specs=[
            pl.BlockSpec((1, RROWS, cin), lambda b, t: (b, t, 0)),
            pl.BlockSpec((1, 1, cin), lambda b, t: (b, 0, 0)),
            pl.BlockSpec((1, 1, cin), lambda b, t: (b, 0, 0)),
            pl.BlockSpec((1, cin), lambda b, t: (0, 0)),
            pl.BlockSpec((1, cin), lambda b, t: (0, 0)),
            pl.BlockSpec((1, tn, 3), lambda b, t: (b, t, 0)),
            pl.BlockSpec((3, cout), lambda b, t: (0, 0)),
            pl.BlockSpec((cout, cin), lambda b, t: (0, 0)),
        ],
        out_specs=[
            pl.BlockSpec((1, tn, cout), lambda b, t: (b, t, 0)),
            pl.BlockSpec((1, tn, cout), lambda b, t: (b, t, 0)),
        ],
        out_shape=[
            jax.ShapeDtypeStruct((NBATCH, NPTS, cout), jnp.float32),
            jax.ShapeDtypeStruct((NBATCH, NPTS, cout), jnp.float32),
        ],
    )(y, s1, s2, gam, bet, xyzt, wxt, wf)


def _pool2_body(y_ref, s1_ref, s2_ref, gam_ref, bet_ref, w1_ref,
                x1_ref, u1_ref, u2_ref, *, mcount):
    b = pl.program_id(0)
    t = pl.program_id(1)
    mean = s1_ref[0] / mcount
    var = s2_ref[0] / mcount - mean * mean
    scale = gam_ref[...] * lax.rsqrt(var + EPS)
    shift = bet_ref[...] - mean * scale
    h = jnp.maximum(y_ref[0] * scale + shift, 0.0)
    c = h.shape[1]
    f = jnp.max(h.reshape(RROWS // KNN, KNN, c), axis=1)
    x1 = lax.dot_general(f, w1_ref[...], (((1,), (1,)), ((), ())),
                         preferred_element_type=jnp.float32)
    x1_ref[0] = x1

    @pl.when((b == 0) & (t == 0))
    def _():
        u1_ref[...] = jnp.zeros_like(u1_ref)
        u2_ref[...] = jnp.zeros_like(u2_ref)

    u1_ref[...] += jnp.sum(x1, axis=0, keepdims=True)
    u2_ref[...] += jnp.sum(x1 * x1, axis=0, keepdims=True)


def _pool2_call(y, s1, s2, gam, bet, w1):
    m = NPTS * KNN
    nt = m // RROWS
    tn = RROWS // KNN
    cin = y.shape[2]
    cout = w1.shape[0]
    body = functools.partial(_pool2_body, mcount=float(m))
    return pl.pallas_call(
        body,
        grid=(NBATCH, nt),
        in_specs=[
            pl.BlockSpec((1, RROWS, cin), lambda b, t: (b, t, 0)),
            pl.BlockSpec((1, 1, cin), lambda b, t: (b, 0, 0)),
            pl.BlockSpec((1, 1, cin), lambda b, t: (b, 0, 0)),
            pl.BlockSpec((1, cin), lambda b, t: (0, 0)),
            pl.BlockSpec((1, cin), lambda b, t: (0, 0)),
            pl.BlockSpec((cout, cin), lambda b, t: (0, 0)),
        ],
        out_specs=[
            pl.BlockSpec((1, tn, cout), lambda b, t: (b, t, 0)),
            pl.BlockSpec((1, cout), lambda b, t: (0, 0)),
            pl.BlockSpec((1, cout), lambda b, t: (0, 0)),
        ],
        out_shape=[
            jax.ShapeDtypeStruct((NBATCH, NPTS, cout), jnp.float32),
            jax.ShapeDtypeStruct((1, cout), jnp.float32),
            jax.ShapeDtypeStruct((1, cout), jnp.float32),
        ],
    )(y, s1, s2, gam, bet, w1)


def _final_body(x_ref, u1_ref, u2_ref, gam_ref, bet_ref, w2_ref, b2_ref,
                q_ref, acc_ref, *, nt):
    t = pl.program_id(1)
    tot = float(NBATCH * NPTS)
    mean = u1_ref[...] / tot
    var = u2_ref[...] / tot - mean * mean
    scale = gam_ref[...] * lax.rsqrt(var + EPS)
    shift = bet_ref[...] - mean * scale
    xh = jnp.maximum(x_ref[0] * scale + shift, 0.0)

    @pl.when(t == 0)
    def _():
        acc_ref[...] = jnp.zeros_like(acc_ref)

    acc_ref[...] += jnp.sum(xh, axis=0, keepdims=True)

    @pl.when(t == nt - 1)
    def _():
        q_ref[...] = (lax.dot_general(
            acc_ref[...], w2_ref[...], (((1,), (1,)), ((), ())),
            preferred_element_type=jnp.float32) / float(NPTS)
            + b2_ref[...]).reshape(1, 1, -1)


def _final_call(x1, u1, u2, gam, bet, w2, b2):
    rt = 2048
    nt = NPTS // rt
    c = x1.shape[2]
    cout = w2.shape[0]
    body = functools.partial(_final_body, nt=nt)
    return pl.pallas_call(
        body,
        grid=(NBATCH, nt),
        in_specs=[
            pl.BlockSpec((1, rt, c), lambda b, t: (b, t, 0)),
            pl.BlockSpec((1, c), lambda b, t: (0, 0)),
            pl.BlockSpec((1, c), lambda b, t: (0, 0)),
            pl.BlockSpec((1, c), lambda b, t: (0, 0)),
            pl.BlockSpec((1, c), lambda b, t: (0, 0)),
            pl.BlockSpec((cout, c), lambda b, t: (0, 0)),
            pl.BlockSpec((1, cout), lambda b, t: (0, 0)),
        ],
        out_specs=pl.BlockSpec((1, 1, cout), lambda b, t: (b, 0, 0)),
        out_shape=jax.ShapeDtypeStruct((NBATCH, 1, cout), jnp.float32),
        scratch_shapes=[pltpu.VMEM((1, c), jnp.float32)],
    )(x1, u1, u2, gam, bet, w2, b2)


def kernel(pc, feature,
           sa1_w0, sa1_g0, sa1_b0, sa1_w1, sa1_g1, sa1_b1,
           sa1_w2, sa1_g2, sa1_b2,
           sa2_w0, sa2_g0, sa2_b0, sa2_w1, sa2_g1, sa2_b1,
           sa2_w2, sa2_g2, sa2_b2,
           conv1_w, bn1_g, bn1_b, conv2_w, conv2_b):
    xyzt = jnp.transpose(pc, (0, 2, 1))
    featt = jnp.transpose(feature, (0, 2, 1))
    wx1t = jnp.transpose(sa1_w0[:, :3])
    wf1t = jnp.transpose(sa1_w0[:, 3:])

    idxg, p1, g1 = _knn_call(xyzt, pc, featt, wx1t, wf1t, sa1_w0.shape[0])

    rows = NBATCH * NPTS * KNN
    idx2d = idxg.reshape(rows // 128, 128)

    c1 = sa1_w0.shape[0]
    gath1 = _sc_gather(g1.reshape(NBATCH * NPTS, c1), idx2d, c1)
    y1, s1a, s2a = _mlpa_call(gath1.reshape(NBATCH, NPTS * KNN, c1), p1, c1)
    y2, s1b, s2b = _layer_call(y1, s1a, s2a, sa1_g0.reshape(1, -1),
                               sa1_b0.reshape(1, -1), sa1_w1)
    y3, s1c, s2c = _layer_call(y2, s1b, s2b, sa1_g1.reshape(1, -1),
                               sa1_b1.reshape(1, -1), sa1_w2)
    wx2t = jnp.transpose(sa2_w0[:, :3])
    wf2 = sa2_w0[:, 3:]
    p2, g2 = _pool1_call(y3, s1c, s2c, sa1_g2.reshape(1, -1),
                         sa1_b2.reshape(1, -1), xyzt, wx2t, wf2)

    c2 = sa2_w0.shape[0]
    gath2 = _sc_gather(g2.reshape(NBATCH * NPTS, c2), idx2d, c2)
    z1, t1a, t2a = _mlpa_call(gath2.reshape(NBATCH, NPTS * KNN, c2), p2, c2)
    z2, t1b, t2b = _layer_call(z1, t1a, t2a, sa2_g0.reshape(1, -1),
                               sa2_b0.reshape(1, -1), sa2_w1)
    z3, t1c, t2c = _layer_call(z2, t1b, t2b, sa2_g1.reshape(1, -1),
                               sa2_b1.reshape(1, -1), sa2_w2)
    x1, u1, u2 = _pool2_call(z3, t1c, t2c, sa2_g2.reshape(1, -1),
                             sa2_b2.reshape(1, -1), conv1_w)

    q = _final_call(x1, u1, u2, bn1_g.reshape(1, -1), bn1_b.reshape(1, -1),
                    conv2_w, conv2_b.reshape(1, -1))
    return q.reshape(NBATCH, conv2_w.shape[0], 1)


# speedup vs baseline: 13.0215x; 2.2383x over previous
import functools

import jax
import jax.numpy as jnp
from jax import lax
from jax.experimental import pallas as pl
from jax.experimental.pallas import tpu as pltpu
from jax.experimental.pallas import tpu_sc as plsc

KNN = 16
EPS = 1e-5
NPTS = 8192
NBATCH = 2
TQ = 32
RROWS = 2048


def _ce(cols, i, j, asc):
    a, b = cols[i], cols[j]
    lo, hi = jnp.minimum(a, b), jnp.maximum(a, b)
    cols[i], cols[j] = (lo, hi) if asc else (hi, lo)


def _bitonic_sort16(cols):
    k = 2
    while k <= 16:
        j = k // 2
        while j >= 1:
            for i in range(16):
                l = i ^ j
                if l > i:
                    _ce(cols, i, l, (i & k) == 0)
            j //= 2
        k *= 2


def _bitonic_merge16(cols):
    for gap in (8, 4, 2, 1):
        for i in range(16):
            if i % (2 * gap) < gap:
                _ce(cols, i, i + gap, True)


def _bottom16(a, b):
    c = [jnp.minimum(a[i], b[15 - i]) for i in range(16)]
    _bitonic_merge16(c)
    return c


def _knn_body(q_ref, kall_ref, f_ref, wx_ref, wf_ref,
              idx_ref, p_ref, g_ref):
    b = pl.program_id(0)
    q = q_ref[0]
    kk = kall_ref[0]
    sqk = jnp.sum(kk * kk, axis=0).reshape(1, NPTS)
    sqq = jnp.sum(q * q, axis=1).reshape(TQ, 1)
    dot = lax.dot_general(q, kk, (((1,), (0,)), ((), ())),
                          preferred_element_type=jnp.float32)
    d = jnp.maximum(sqk + sqq - 2.0 * dot, 0.0)
    keys = (lax.bitcast_convert_type(d, jnp.int32) & jnp.int32(-8192)) | \
        lax.broadcasted_iota(jnp.int32, (TQ, NPTS), 1)
    slabs = [keys[:, 128 * g:128 * (g + 1)] for g in range(NPTS // 128)]
    quads = []
    for g4 in range(4):
        grp = slabs[16 * g4:16 * (g4 + 1)]
        _bitonic_sort16(grp)
        quads.append(grp)
    m1 = _bottom16(quads[0], quads[1])
    m2 = _bottom16(quads[2], quads[3])
    cand = jnp.concatenate(
        [jnp.minimum(m1[i], m2[15 - i]) for i in range(16)], axis=1)
    cols = []
    bound = jnp.full((TQ, 1), -1, jnp.int32)
    for _ in range(KNN):
        masked = jnp.where(cand > bound, cand, jnp.int32(2**31 - 1))
        bound = jnp.min(masked, axis=1, keepdims=True)
        cols.append(bound)
    packed = jnp.concatenate(cols, axis=1)
    idx_ref[0] = (packed & jnp.int32(8191)) + b * NPTS
    p1 = lax.dot_general(q, wx_ref[...], (((1,), (0,)), ((), ())),
                         preferred_element_type=jnp.float32)
    g1 = p1 + lax.dot_general(f_ref[0], wf_ref[...], (((1,), (0,)), ((), ())),
                              preferred_element_type=jnp.float32)
    p_ref[0] = p1
    g_ref[0] = g1


def _knn_call(xyzt, pc, featt, wxt, wft, cout):
    nt = NPTS // TQ
    return pl.pallas_call(
        _knn_body,
        grid=(NBATCH, nt),
        in_specs=[
            pl.BlockSpec((1, TQ, 3), lambda b, t: (b, t, 0)),
            pl.BlockSpec((1, 3, NPTS), lambda b, t: (b, 0, 0)),
            pl.BlockSpec((1, TQ, 3), lambda b, t: (b, t, 0)),
            pl.BlockSpec((3, cout), lambda b, t: (0, 0)),
            pl.BlockSpec((3, cout), lambda b, t: (0, 0)),
        ],
        out_specs=[
            pl.BlockSpec((1, TQ, KNN), lambda b, t: (b, t, 0)),
            pl.BlockSpec((1, TQ, cout), lambda b, t: (b, t, 0)),
            pl.BlockSpec((1, TQ, cout), lambda b, t: (b, t, 0)),
        ],
        out_shape=[
            jax.ShapeDtypeStruct((NBATCH, NPTS, KNN), jnp.int32),
            jax.ShapeDtypeStruct((NBATCH, NPTS, cout), jnp.float32),
            jax.ShapeDtypeStruct((NBATCH, NPTS, cout), jnp.float32),
        ],
    )(xyzt, pc, featt, wxt, wft)


def _sc_gather(table, idx2d, cwidth):
    rows = idx2d.shape[0] * 128
    nworkers = 32
    per_w = rows // nworkers
    nch = per_w // 128
    mesh = plsc.VectorSubcoreMesh(core_axis_name="c", subcore_axis_name="s")

    @functools.partial(
        pl.kernel,
        mesh=mesh,
        out_type=jax.ShapeDtypeStruct((rows, cwidth), jnp.float32),
        compiler_params=pltpu.CompilerParams(use_tc_tiling_on_sc=False),
        scratch_types=[
            pltpu.VMEM((nch, 128), jnp.int32),
            pltpu.VMEM((128, cwidth), jnp.float32),
            pltpu.SemaphoreType.DMA,
        ],
    )
    def gather_k(table_hbm, idx_hbm, out_hbm, idx_v, rows_v, sem):
        wid = lax.axis_index("s") * 2 + lax.axis_index("c")
        pltpu.sync_copy(idx_hbm.at[pl.ds(wid * nch, nch)], idx_v)

        def step(j, carry):
            pltpu.async_copy(table_hbm.at[idx_v.at[j]], rows_v, sem).wait()
            pltpu.sync_copy(rows_v,
                            out_hbm.at[pl.ds((wid * nch + j) * 128, 128)])
            return carry

        lax.fori_loop(0, nch, step, 0)

    return gather_k(table, idx2d)


def _mlpa_body(g_ref, p_ref, y_ref, t1_ref, t2_ref):
    t = pl.program_id(1)
    g = g_ref[0]
    p = p_ref[0]
    c = g.shape[1]
    y = (g.reshape(RROWS // KNN, KNN, c) - p[:, None, :]).reshape(RROWS, c)
    y_ref[0] = y

    @pl.when(t == 0)
    def _():
        t1_ref[...] = jnp.zeros_like(t1_ref)
        t2_ref[...] = jnp.zeros_like(t2_ref)

    t1_ref[...] += jnp.sum(y, axis=0).reshape(1, 1, -1)
    t2_ref[...] += jnp.sum(y * y, axis=0).reshape(1, 1, -1)


def _mlpa_call(gathered, pmat, c):
    m = NPTS * KNN
    nt = m // RROWS
    return pl.pallas_call(
        _mlpa_body,
        grid=(NBATCH, nt),
        in_specs=[
            pl.BlockSpec((1, RROWS, c), lambda b, t: (b, t, 0)),
            pl.BlockSpec((1, RROWS // KNN, c), lambda b, t: (b, t, 0)),
        ],
        out_specs=[
            pl.BlockSpec((1, RROWS, c), lambda b, t: (b, t, 0)),
            pl.BlockSpec((1, 1, c), lambda b, t: (b, 0, 0)),
            pl.BlockSpec((1, 1, c), lambda b, t: (b, 0, 0)),
        ],
        out_shape=[
            jax.ShapeDtypeStruct((NBATCH, m, c), jnp.float32),
            jax.ShapeDtypeStruct((NBATCH, 1, c), jnp.float32),
            jax.ShapeDtypeStruct((NBATCH, 1, c), jnp.float32),
        ],
    )(gathered, pmat)


def _layer_body(y_ref, s1_ref, s2_ref, gam_ref, bet_ref, w_ref,
                o_ref, t1_ref, t2_ref, *, mcount):
    t = pl.program_id(1)
    mean = s1_ref[0] / mcount
    var = s2_ref[0] / mcount - mean * mean
    scale = gam_ref[...] * lax.rsqrt(var + EPS)
    shift = bet_ref[...] - mean * scale
    h = jnp.maximum(y_ref[0] * scale + shift, 0.0)
    ynew = lax.dot_general(h, w_ref[...], (((1,), (1,)), ((), ())),
                           preferred_element_type=jnp.float32)
    o_ref[0] = ynew

    @pl.when(t == 0)
    def _():
        t1_ref[...] = jnp.zeros_like(t1_ref)
        t2_ref[...] = jnp.zeros_like(t2_ref)

    t1_ref[...] += jnp.sum(ynew, axis=0).reshape(1, 1, -1)
    t2_ref[...] += jnp.sum(ynew * ynew, axis=0).reshape(1, 1, -1)


def _layer_call(y, s1, s2, gam, bet, w):
    m = NPTS * KNN
    nt = m // RROWS
    cin = y.shape[2]
    cout = w.shape[0]
    body = functools.partial(_layer_body, mcount=float(m))
    return pl.pallas_call(
        body,
        grid=(NBATCH, nt),
        in_specs=[
            pl.BlockSpec((1, RROWS, cin), lambda b, t: (b, t, 0)),
            pl.BlockSpec((1, 1, cin), lambda b, t: (b, 0, 0)),
            pl.BlockSpec((1, 1, cin), lambda b, t: (b, 0, 0)),
            pl.BlockSpec((1, cin), lambda b, t: (0, 0)),
            pl.BlockSpec((1, cin), lambda b, t: (0, 0)),
            pl.BlockSpec((cout, cin), lambda b, t: (0, 0)),
        ],
        out_specs=[
            pl.BlockSpec((1, RROWS, cout), lambda b, t: (b, t, 0)),
            pl.BlockSpec((1, 1, cout), lambda b, t: (b, 0, 0)),
            pl.BlockSpec((1, 1, cout), lambda b, t: (b, 0, 0)),
        ],
        out_shape=[
            jax.ShapeDtypeStruct((NBATCH, m, cout), jnp.float32),
            jax.ShapeDtypeStruct((NBATCH, 1, cout), jnp.float32),
            jax.ShapeDtypeStruct((NBATCH, 1, cout), jnp.float32),
        ],
    )(y, s1, s2, gam, bet, w)


def _pool1_body(y_ref, s1_ref, s2_ref, gam_ref, bet_ref, xyz_ref,
                wx_ref, wf_ref, p2_ref, g2_ref, *, mcount):
    mean = s1_ref[0] / mcount
    var = s2_ref[0] / mcount - mean * mean
    scale = gam_ref[...] * lax.rsqrt(var + EPS)
    shift = bet_ref[...] - mean * scale
    h = jnp.maximum(y_ref[0] * scale + shift, 0.0)
    c = h.shape[1]
    f = jnp.max(h.reshape(RROWS // KNN, KNN, c), axis=1)
    p2 = lax.dot_general(xyz_ref[0], wx_ref[...], (((1,), (0,)), ((), ())),
                         preferred_element_type=jnp.float32)
    g2 = p2 + lax.dot_general(f, wf_ref[...], (((1,), (1,)), ((), ())),
                              preferred_element_type=jnp.float32)
    p2_ref[0] = p2
    g2_ref[0] = g2


def _pool1_call(y, s1, s2, gam, bet, xyzt, wxt, wf):
    m = NPTS * KNN
    nt = m // RROWS
    tn = RROWS // KNN
    cin = y.shape[2]
    cout = wf.shape[0]
    body = functools.partial(_pool1_body, mcount=float(m))
    return pl.pallas_call(
        body,
        grid=(NBATCH, nt),
        in_specs=[
            pl.BlockSpec((1, RROWS, cin), lambda b, t: (b, t, 0)),
            pl.BlockSpec((1, 1, cin), lambda b, t: (b, 0, 0)),
            pl.BlockSpec((1, 1, cin), lambda b, t: (b, 0, 0)),
            pl.BlockSpec((1, cin), lambda b, t: (0, 0)),
            pl.BlockSpec((1, cin), lambda b, t: (0, 0)),
            pl.BlockSpec((1, tn, 3), lambda b, t: (b, t, 0)),
            pl.BlockSpec((3, cout), lambda b, t: (0, 0)),
            pl.BlockSpec((cout, cin), lambda b, t: (0, 0)),
        ],
        out_specs=[
            pl.BlockSpec((1, tn, cout), lambda b, t: (b, t, 0)),
            pl.BlockSpec((1, tn, cout), lambda b, t: (b, t, 0)),
        ],
        out_shape=[
            jax.ShapeDtypeStruct((NBATCH, NPTS, cout), jnp.float32),
            jax.ShapeDtypeStruct((NBATCH, NPTS, cout), jnp.float32),
        ],
    )(y, s1, s2, gam, bet, xyzt, wxt, wf)


def _pool2_body(y_ref, s1_ref, s2_ref, gam_ref, bet_ref, w1_ref,
                x1_ref, u1_ref, u2_ref, *, mcount):
    b = pl.program_id(0)
    t = pl.program_id(1)
    mean = s1_ref[0] / mcount
    var = s2_ref[0] / mcount - mean * mean
    scale = gam_ref[...] * lax.rsqrt(var + EPS)
    shift = bet_ref[...] - mean * scale
    h = jnp.maximum(y_ref[0] * scale + shift, 0.0)
    c = h.shape[1]
    f = jnp.max(h.reshape(RROWS // KNN, KNN, c), axis=1)
    x1 = lax.dot_general(f, w1_ref[...], (((1,), (1,)), ((), ())),
                         preferred_element_type=jnp.float32)
    x1_ref[0] = x1

    @pl.when((b == 0) & (t == 0))
    def _():
        u1_ref[...] = jnp.zeros_like(u1_ref)
        u2_ref[...] = jnp.zeros_like(u2_ref)

    u1_ref[...] += jnp.sum(x1, axis=0, keepdims=True)
    u2_ref[...] += jnp.sum(x1 * x1, axis=0, keepdims=True)


def _pool2_call(y, s1, s2, gam, bet, w1):
    m = NPTS * KNN
    nt = m // RROWS
    tn = RROWS // KNN
    cin = y.shape[2]
    cout = w1.shape[0]
    body = functools.partial(_pool2_body, mcount=float(m))
    return pl.pallas_call(
        body,
        grid=(NBATCH, nt),
        in_specs=[
            pl.BlockSpec((1, RROWS, cin), lambda b, t: (b, t, 0)),
            pl.BlockSpec((1, 1, cin), lambda b, t: (b, 0, 0)),
            pl.BlockSpec((1, 1, cin), lambda b, t: (b, 0, 0)),
            pl.BlockSpec((1, cin), lambda b, t: (0, 0)),
            pl.BlockSpec((1, cin), lambda b, t: (0, 0)),
            pl.BlockSpec((cout, cin), lambda b, t: (0, 0)),
        ],
        out_specs=[
            pl.BlockSpec((1, tn, cout), lambda b, t: (b, t, 0)),
            pl.BlockSpec((1, cout), lambda b, t: (0, 0)),
            pl.BlockSpec((1, cout), lambda b, t: (0, 0)),
        ],
        out_shape=[
            jax.ShapeDtypeStruct((NBATCH, NPTS, cout), jnp.float32),
            jax.ShapeDtypeStruct((1, cout), jnp.float32),
            jax.ShapeDtypeStruct((1, cout), jnp.float32),
        ],
    )(y, s1, s2, gam, bet, w1)


def _final_body(x_ref, u1_ref, u2_ref, gam_ref, bet_ref, w2_ref, b2_ref,
                q_ref, acc_ref, *, nt):
    t = pl.program_id(1)
    tot = float(NBATCH * NPTS)
    mean = u1_ref[...] / tot
    var = u2_ref[...] / tot - mean * mean
    scale = gam_ref[...] * lax.rsqrt(var + EPS)
    shift = bet_ref[...] - mean * scale
    xh = jnp.maximum(x_ref[0] * scale + shift, 0.0)

    @pl.when(t == 0)
    def _():
        acc_ref[...] = jnp.zeros_like(acc_ref)

    acc_ref[...] += jnp.sum(xh, axis=0, keepdims=True)

    @pl.when(t == nt - 1)
    def _():
        q_ref[...] = (lax.dot_general(
            acc_ref[...], w2_ref[...], (((1,), (1,)), ((), ())),
            preferred_element_type=jnp.float32) / float(NPTS)
            + b2_ref[...]).reshape(1, 1, -1)


def _final_call(x1, u1, u2, gam, bet, w2, b2):
    rt = 2048
    nt = NPTS // rt
    c = x1.shape[2]
    cout = w2.shape[0]
    body = functools.partial(_final_body, nt=nt)
    return pl.pallas_call(
        body,
        grid=(NBATCH, nt),
        in_specs=[
            pl.BlockSpec((1, rt, c), lambda b, t: (b, t, 0)),
            pl.BlockSpec((1, c), lambda b, t: (0, 0)),
            pl.BlockSpec((1, c), lambda b, t: (0, 0)),
            pl.BlockSpec((1, c), lambda b, t: (0, 0)),
            pl.BlockSpec((1, c), lambda b, t: (0, 0)),
            pl.BlockSpec((cout, c), lambda b, t: (0, 0)),
            pl.BlockSpec((1, cout), lambda b, t: (0, 0)),
        ],
        out_specs=pl.BlockSpec((1, 1, cout), lambda b, t: (b, 0, 0)),
        out_shape=jax.ShapeDtypeStruct((NBATCH, 1, cout), jnp.float32),
        scratch_shapes=[pltpu.VMEM((1, c), jnp.float32)],
    )(x1, u1, u2, gam, bet, w2, b2)


def kernel(pc, feature,
           sa1_w0, sa1_g0, sa1_b0, sa1_w1, sa1_g1, sa1_b1,
           sa1_w2, sa1_g2, sa1_b2,
           sa2_w0, sa2_g0, sa2_b0, sa2_w1, sa2_g1, sa2_b1,
           sa2_w2, sa2_g2, sa2_b2,
           conv1_w, bn1_g, bn1_b, conv2_w, conv2_b):
    xyzt = jnp.transpose(pc, (0, 2, 1))
    featt = jnp.transpose(feature, (0, 2, 1))
    wx1t = jnp.transpose(sa1_w0[:, :3])
    wf1t = jnp.transpose(sa1_w0[:, 3:])

    idxg, p1, g1 = _knn_call(xyzt, pc, featt, wx1t, wf1t, sa1_w0.shape[0])

    rows = NBATCH * NPTS * KNN
    idx2d = idxg.reshape(rows // 128, 128)

    c1 = sa1_w0.shape[0]
    gath1 = _sc_gather(g1.reshape(NBATCH * NPTS, c1), idx2d, c1)
    y1, s1a, s2a = _mlpa_call(gath1.reshape(NBATCH, NPTS * KNN, c1), p1, c1)
    y2, s1b, s2b = _layer_call(y1, s1a, s2a, sa1_g0.reshape(1, -1),
                               sa1_b0.reshape(1, -1), sa1_w1)
    y3, s1c, s2c = _layer_call(y2, s1b, s2b, sa1_g1.reshape(1, -1),
                               sa1_b1.reshape(1, -1), sa1_w2)
    wx2t = jnp.transpose(sa2_w0[:, :3])
    wf2 = sa2_w0[:, 3:]
    p2, g2 = _pool1_call(y3, s1c, s2c, sa1_g2.reshape(1, -1),
                         sa1_b2.reshape(1, -1), xyzt, wx2t, wf2)

    c2 = sa2_w0.shape[0]
    gath2 = _sc_gather(g2.reshape(NBATCH * NPTS, c2), idx2d, c2)
    z1, t1a, t2a = _mlpa_call(gath2.reshape(NBATCH, NPTS * KNN, c2), p2, c2)
    z2, t1b, t2b = _layer_call(z1, t1a, t2a, sa2_g0.reshape(1, -1),
                               sa2_b0.reshape(1, -1), sa2_w1)
    z3, t1c, t2c = _layer_call(z2, t1b, t2b, sa2_g1.reshape(1, -1),
                               sa2_b1.reshape(1, -1), sa2_w2)
    x1, u1, u2 = _pool2_call(z3, t1c, t2c, sa2_g2.reshape(1, -1),
                             sa2_b2.reshape(1, -1), conv1_w)

    q = _final_call(x1, u1, u2, bn1_g.reshape(1, -1), bn1_b.reshape(1, -1),
                    conv2_w, conv2_b.reshape(1, -1))
    return q.reshape(NBATCH, conv2_w.shape[0], 1)
